# Initial kernel scaffold; baseline (speedup 1.0000x reference)
import jax
import jax.numpy as jnp
from jax.experimental import pallas as pl


def kernel(x, pos, Wf1, bf1, Wf2, bf2, Wm1, bm1, Wlast, blast, Wg, bg):
    raise NotImplementedError("write your pallas kernel here")


# speedup vs baseline: 16.9494x; 16.9494x over previous
import functools

import jax
import jax.numpy as jnp
from jax import lax
from jax.experimental import pallas as pl
from jax.experimental.pallas import tpu as pltpu
from jax.experimental.pallas import tpu_sc as plsc

_K = 16
_BIG = 3.0e38
_PAD_COORD = 1.0e6


def _knn_body(posq_ref, post_ref, sqall_ref, out_ref):
    posq = posq_ref[...]
    s = jnp.dot(posq, post_ref[...], preferred_element_type=jnp.float32)
    sqq = jnp.sum(posq * posq, axis=1, keepdims=True)
    d = sqq + sqall_ref[...] - 2.0 * s
    col = lax.broadcasted_iota(jnp.int32, d.shape, 1).astype(jnp.float32)
    cols = []
    for k in range(_K + 1):
        m = jnp.min(d, axis=1, keepdims=True)
        idxf = jnp.min(jnp.where(d == m, col, 1e9),
                       axis=1, keepdims=True)
        if k >= 1:
            cols.append(idxf)
        d = jnp.where(col == idxf, _BIG, d)
    out_ref[...] = jnp.concatenate(cols, axis=1).astype(jnp.int32)


def _knn_topk(posp, sqall, tq):
    np_, _ = posp.shape
    grid = np_ // tq
    return pl.pallas_call(
        _knn_body,
        grid=(grid,),
        in_specs=[
            pl.BlockSpec((tq, 8), lambda i: (i, 0)),
            pl.BlockSpec((8, np_), lambda i: (0, 0)),
            pl.BlockSpec((1, np_), lambda i: (0, 0)),
        ],
        out_specs=pl.BlockSpec((tq, _K), lambda i: (i, 0)),
        out_shape=jax.ShapeDtypeStruct((np_, _K), jnp.int32),
    )(posp, posp.T, sqall)


_NC, _NS = 2, 16
_NW = _NC * _NS
_CH = 128


def _sc_gather(table, idx_flat):
    total = idx_flat.shape[0]
    d = table.shape[1]
    b_per_w = total // _NW
    n_chunks = b_per_w // _CH
    mesh = plsc.VectorSubcoreMesh(core_axis_name="c", subcore_axis_name="s")

    @functools.partial(
        pl.kernel,
        out_type=jax.ShapeDtypeStruct((total, d), jnp.float32),
        mesh=mesh,
        scratch_types=[
            pltpu.VMEM((_CH,), jnp.int32),
            pltpu.VMEM((_CH, d), jnp.float32),
            pltpu.SemaphoreType.DMA,
        ],
        compiler_params=pltpu.CompilerParams(use_tc_tiling_on_sc=False),
    )
    def k(table_hbm, idx_hbm, out_hbm, idx_v, rows_v, sem):
        wid = lax.axis_index("s") * _NC + lax.axis_index("c")
        base = wid * b_per_w

        def body(g, carry):
            off = base + g * _CH
            pltpu.sync_copy(idx_hbm.at[pl.ds(off, _CH)], idx_v)
            pltpu.async_copy(table_hbm.at[idx_v], rows_v, sem).wait()
            pltpu.sync_copy(rows_v, out_hbm.at[pl.ds(off, _CH)])
            return carry

        lax.fori_loop(0, n_chunks, body, 0)

    return k(table, idx_flat)


def _mlp_body(x_ref, kf_ref, a1_ref, bf1_ref, b1_ref, wf2_ref, bf2_ref,
              wm1a_ref, wm1b_ref, bm1_ref, wg_ref, bg_ref, wlast_ref,
              blast_ref, out_ref):
    tp = x_ref.shape[0]
    xv = x_ref[...]
    kf = kf_ref[...]
    f32 = jnp.float32
    hx = jnp.dot(xv, a1_ref[...], preferred_element_type=f32) + bf1_ref[...]
    hk = jnp.dot(kf, b1_ref[...], preferred_element_type=f32)
    h3 = jnp.maximum(hk.reshape(tp, _K, 256) + hx[:, None, :], 0.0)
    f2 = jnp.maximum(
        jnp.dot(h3.reshape(tp * _K, 256), wf2_ref[...],
                preferred_element_type=f32) + bf2_ref[...], 0.0)
    mx = jnp.dot(xv, wm1b_ref[...], preferred_element_type=f32) + bm1_ref[...]
    mo3 = jnp.maximum(
        jnp.dot(f2, wm1a_ref[...], preferred_element_type=f32)
        .reshape(tp, _K, 32) + mx[:, None, :], 0.0)
    f3 = f2.reshape(tp, _K, 32)
    gf = jnp.concatenate(
        [jnp.mean(mo3, axis=1), jnp.mean(f3, axis=1), xv], axis=-1)
    gl = jnp.dot(gf, wg_ref[...], preferred_element_type=f32) + bg_ref[...]
    gw = jax.nn.sigmoid(gl)
    y3 = jnp.concatenate(
        [mo3, f3, jnp.broadcast_to(xv[:, None, :], (tp, _K, 64))], axis=-1)
    yg = (y3 * gw[:, None, :]).reshape(tp * _K, 128)
    ol = (jnp.dot(yg, wlast_ref[...], preferred_element_type=f32)
          + blast_ref[...])
    o1 = jnp.max(ol.reshape(tp, _K, 32), axis=1)
    o2 = gw * jnp.concatenate(
        [jnp.max(mo3, axis=1), jnp.max(f3, axis=1), xv], axis=-1)
    out_ref[...] = jnp.concatenate([o1, o2], axis=-1)


def _mlp(xp, kf, weights, tp):
    np_ = xp.shape[0]
    grid = np_ // tp
    wspecs = [pl.BlockSpec(w.shape, lambda i: tuple(0 for _ in w.shape))
              for w in weights]
    return pl.pallas_call(
        _mlp_body,
        grid=(grid,),
        in_specs=[
            pl.BlockSpec((tp, 64), lambda i: (i, 0)),
            pl.BlockSpec((tp * _K, 64), lambda i: (i, 0)),
        ] + wspecs,
        out_specs=pl.BlockSpec((tp, 160), lambda i: (i, 0)),
        out_shape=jax.ShapeDtypeStruct((np_, 160), jnp.float32),
    )(xp, kf, *weights)


def kernel(x, pos, Wf1, bf1, Wf2, bf2, Wm1, bm1, Wlast, blast, Wg, bg):
    _, n, d = x.shape
    x2 = x[0]
    pos2 = pos[0]
    tq, tp = 256, 128
    np_ = ((n + tq - 1) // tq) * tq

    posp = jnp.zeros((np_, 8), jnp.float32)
    posp = posp.at[:n, :3].set(pos2)
    posp = posp.at[n:, :3].set(_PAD_COORD)
    sq = jnp.sum(pos2 * pos2, axis=-1)
    sqall = jnp.concatenate(
        [sq, jnp.full((np_ - n,), 3.0 * _PAD_COORD * _PAD_COORD, jnp.float32)]
    )[None, :]

    idx = _knn_topk(posp, sqall, tq)
    idx_flat = idx[:n].reshape(-1)
    idx_flat = jnp.concatenate(
        [idx_flat, jnp.zeros((np_ * _K - n * _K,), jnp.int32)])

    kf = _sc_gather(x2, idx_flat)

    g = Wf1[2 * d:]
    weights = [
        Wf1[:d] - g,
        bf1[None, :],
        Wf1[d:2 * d] + g,
        Wf2,
        bf2[None, :],
        Wm1[:32],
        Wm1[32:],
        bm1[None, :],
        Wg,
        bg[None, :],
        Wlast,
        blast[None, :],
    ]
    xp = jnp.zeros((np_, d), jnp.float32).at[:n].set(x2)
    out = _mlp(xp, kf, weights, tp)
    return out[:n][None]
